# Initial kernel scaffold; baseline (speedup 1.0000x reference)
import jax
import jax.numpy as jnp
from jax.experimental import pallas as pl


def kernel(x, edge_index, batch, Wq0, bq0, Wk0, bk0, Wv0, bv0, Ws0, bs0, Wq1, bq1, Wk1, bk1, Wv1, bv1, Ws1, bs1, linW, linb):
    raise NotImplementedError("write your pallas kernel here")


# speedup vs baseline: 11.2198x; 11.2198x over previous
import dataclasses
import functools
import math

import jax
import jax.numpy as jnp
from jax import lax
from jax.experimental import pallas as pl
from jax.experimental.pallas import tpu as pltpu
from jax.experimental.pallas import tpu_sc as plsc

N, E, D, H, C, G, NC = 10000, 320000, 128, 1, 128, 128, 10


_BN = 1000
_NBLK = N // _BN


def _qkvs_body(x_ref, w_ref, b_ref, q_ref, k_ref, v_ref, s_ref):
    acc = jnp.dot(x_ref[...], w_ref[...], preferred_element_type=jnp.float32)
    acc = acc + b_ref[...]
    q_ref[...] = acc[:, 0:128]
    k_ref[...] = acc[:, 128:256]
    v_ref[...] = acc[:, 256:384]
    s_ref[...] = acc[:, 384:512]


def _tc_qkvs(x, wcat, bcat):
    out = jax.ShapeDtypeStruct((N, 128), jnp.float32)
    return pl.pallas_call(
        _qkvs_body,
        grid=(_NBLK,),
        in_specs=[
            pl.BlockSpec((_BN, 128), lambda i: (i, 0)),
            pl.BlockSpec((128, 512), lambda i: (0, 0)),
            pl.BlockSpec((1, 512), lambda i: (0, 0)),
        ],
        out_specs=[pl.BlockSpec((_BN, 128), lambda i: (i, 0))] * 4,
        out_shape=[out, out, out, out],
    )(x, wcat, bcat)


def _combine(p_ref, d_ref, s_ref):
    den = jnp.sum(d_ref[0], axis=0)
    num = p_ref[0] + p_ref[1]
    h = num / (den[:, None] + 1e-16) + s_ref[...]
    return jnp.maximum(h, 0.0)


def _mid_body(p_ref, d_ref, s_ref, w_ref, b_ref, q_ref, k_ref, v_ref, s_out):
    h = _combine(p_ref, d_ref, s_ref)
    acc = jnp.dot(h, w_ref[...], preferred_element_type=jnp.float32)
    acc = acc + b_ref[...]
    q_ref[...] = acc[:, 0:128]
    k_ref[...] = acc[:, 128:256]
    v_ref[...] = acc[:, 256:384]
    s_out[...] = acc[:, 384:512]


def _tc_mid(p, d, s, wcat, bcat):
    out = jax.ShapeDtypeStruct((N, 128), jnp.float32)
    return pl.pallas_call(
        _mid_body,
        grid=(_NBLK,),
        in_specs=[
            pl.BlockSpec((2, _BN, 128), lambda i: (0, i, 0)),
            pl.BlockSpec((1, 32, _BN), lambda i: (i, 0, 0)),
            pl.BlockSpec((_BN, 128), lambda i: (i, 0)),
            pl.BlockSpec((128, 512), lambda i: (0, 0)),
            pl.BlockSpec((1, 512), lambda i: (0, 0)),
        ],
        out_specs=[pl.BlockSpec((_BN, 128), lambda i: (i, 0))] * 4,
        out_shape=[out, out, out, out],
    )(p, d, s, wcat, bcat)


def _final_body(p_ref, d_ref, s_ref, b_ref, w_ref, lb_ref, o_ref, acc_ref):
    i = pl.program_id(0)

    @pl.when(i == 0)
    def _():
        acc_ref[...] = jnp.zeros_like(acc_ref)

    h = _combine(p_ref, d_ref, s_ref)
    bvec = b_ref[0, 0]
    oh = (bvec[:, None] == lax.broadcasted_iota(jnp.int32, (_BN, G), 1))
    oh = oh.astype(jnp.float32)
    sums = lax.dot_general(oh, h, (((0,), (0,)), ((), ())),
                           preferred_element_type=jnp.float32)
    ones = jnp.ones((_BN, 128), jnp.float32)
    cnts = lax.dot_general(oh, ones, (((0,), (0,)), ((), ())),
                           preferred_element_type=jnp.float32)
    acc_ref[:, 0:128] += sums
    acc_ref[:, 128:256] += cnts

    @pl.when(i == _NBLK - 1)
    def _():
        pooled = acc_ref[:, 0:128] / jnp.maximum(acc_ref[:, 128:256], 1.0)
        o_ref[...] = jnp.dot(pooled, w_ref[...],
                             preferred_element_type=jnp.float32) + lb_ref[...]


def _tc_final(p, d, s, batch2d, linw_pad, linb_pad):
    return pl.pallas_call(
        _final_body,
        grid=(_NBLK,),
        in_specs=[
            pl.BlockSpec((2, _BN, 128), lambda i: (0, i, 0)),
            pl.BlockSpec((1, 32, _BN), lambda i: (i, 0, 0)),
            pl.BlockSpec((_BN, 128), lambda i: (i, 0)),
            pl.BlockSpec((1, 1, _BN), lambda i: (i, 0, 0)),
            pl.BlockSpec((128, 128), lambda i: (0, 0)),
            pl.BlockSpec((1, 128), lambda i: (0, 0)),
        ],
        out_specs=pl.BlockSpec((G, 128), lambda i: (0, 0)),
        out_shape=jax.ShapeDtypeStruct((G, 128), jnp.float32),
        scratch_shapes=[pltpu.VMEM((G, 256), jnp.float32)],
    )(p, d, s, batch2d, linw_pad, linb_pad)


_W = 80
_EPT = E // 32
_NCH = _EPT // _W
_RPT = N // 16
_ZR = 125
_INV_SQRT_C = 1.0 / math.sqrt(C)

_sc_mesh = plsc.VectorSubcoreMesh(core_axis_name="c", subcore_axis_name="s")

_sc_cp = pltpu.CompilerParams()
if "needs_layout_passes" in pltpu.CompilerParams.__dataclass_fields__:
    _sc_cp = dataclasses.replace(_sc_cp, needs_layout_passes=False)


@functools.partial(
    pl.kernel,
    out_type=(
        jax.ShapeDtypeStruct((2 * N, 128), jnp.float32),
        jax.ShapeDtypeStruct((32 * N,), jnp.float32),
    ),
    mesh=_sc_mesh,
    compiler_params=_sc_cp,
    scratch_types=[
        pltpu.VMEM((_W,), jnp.int32),
        pltpu.VMEM((_W,), jnp.int32),
        pltpu.VMEM((_W, 128), jnp.float32),
        pltpu.VMEM((_W, 128), jnp.float32),
        pltpu.VMEM((_W, 128), jnp.float32),
        pltpu.VMEM((N,), jnp.float32),
        pltpu.VMEM_SHARED((N, 128), jnp.float32),
        pltpu.SemaphoreType.DMA,
        pltpu.SemaphoreType.DMA,
        pltpu.SemaphoreType.DMA,
    ],
)
def _sc_edge(q_hbm, k_hbm, v_hbm, src_hbm, dst_hbm, p_out, d_out,
             src_v, dst_v, qbuf, kbuf, vbuf, den_tile,
             sp_out, sem0, sem1, sem2):
    c = lax.axis_index("c")
    s = lax.axis_index("s")
    tile = c * 16 + s
    zeros16 = jnp.zeros((16,), jnp.float32)

    @pl.loop(0, N, step=16)
    def _(i):
        den_tile[pl.ds(i, 16)] = zeros16

    @pl.loop(0, 8)
    def _(r):
        @pl.loop(0, 128, step=16)
        def _(jj):
            vbuf[r, pl.ds(jj, 16)] = zeros16

    @pl.loop(0, 78)
    def _(t):
        b = pl.multiple_of(s * 624 + t * 8, 8)
        pltpu.sync_copy(vbuf.at[pl.ds(0, 8)], sp_out.at[pl.ds(b, 8)])

    @pl.when(s == 15)
    def _():
        @pl.loop(0, 2)
        def _(t):
            b = pl.multiple_of(9984 + t * 8, 8)
            pltpu.sync_copy(vbuf.at[pl.ds(0, 8)], sp_out.at[pl.ds(b, 8)])

    plsc.subcore_barrier()

    lanes = lax.iota(jnp.int32, 16)

    @pl.loop(0, _NCH)
    def _(g):
        base = pl.multiple_of(tile * _EPT + g * _W, 16)
        pltpu.sync_copy(src_hbm.at[pl.ds(base, _W)], src_v)
        pltpu.sync_copy(dst_hbm.at[pl.ds(base, _W)], dst_v)
        cq = pltpu.async_copy(q_hbm.at[dst_v], qbuf, sem0)
        ck = pltpu.async_copy(k_hbm.at[src_v], kbuf, sem1)
        cv = pltpu.async_copy(v_hbm.at[src_v], vbuf, sem2)
        cq.wait()
        ck.wait()
        cv.wait()
        @pl.loop(0, _W)
        def _(e):
            p = qbuf[e, pl.ds(0, 16)] * kbuf[e, pl.ds(0, 16)]
            for j in range(1, 8):
                p = p + qbuf[e, pl.ds(j * 16, 16)] * kbuf[e, pl.ds(j * 16, 16)]
            qbuf[e, pl.ds(0, 16)] = p

        @pl.loop(0, _W // 16)
        def _(gi):
            eidx = lanes + gi * 16
            acc = plsc.load_gather(qbuf, [eidx, jnp.zeros((16,), jnp.int32)])
            for l in range(1, 16):
                acc = acc + plsc.load_gather(
                    qbuf, [eidx, jnp.full((16,), l, jnp.int32)])
            ex = jnp.exp(acc * _INV_SQRT_C)
            dst_g = dst_v[pl.ds(gi * 16, 16)]
            plsc.addupdate_scatter(den_tile, [dst_g], ex)
            for l in range(16):
                e = gi * 16 + l
                sc = ex[l]
                for j in range(8):
                    vbuf[e, pl.ds(j * 16, 16)] = vbuf[e, pl.ds(j * 16, 16)] * sc
        pltpu.sync_copy(vbuf, sp_out.at[dst_v], add=True)

    plsc.subcore_barrier()

    @pl.when(s == 0)
    def _():
        pltpu.sync_copy(sp_out, p_out.at[pl.ds(pl.multiple_of(c * N, 16), N)])

    pltpu.sync_copy(den_tile,
                    d_out.at[pl.ds(pl.multiple_of(tile * N, 16), N)])


def kernel(x, edge_index, batch, Wq0, bq0, Wk0, bk0, Wv0, bv0, Ws0, bs0,
           Wq1, bq1, Wk1, bk1, Wv1, bv1, Ws1, bs1, linW, linb):
    src = edge_index[0]
    dst = edge_index[1]
    wc0 = jnp.concatenate([Wq0, Wk0, Wv0, Ws0], axis=1)
    bc0 = jnp.concatenate([bq0, bk0, bv0, bs0])[None, :]
    wc1 = jnp.concatenate([Wq1, Wk1, Wv1, Ws1], axis=1)
    bc1 = jnp.concatenate([bq1, bk1, bv1, bs1])[None, :]
    linw_pad = jnp.pad(linW, ((0, 0), (0, 128 - NC)))
    linb_pad = jnp.pad(linb, (0, 128 - NC))[None, :]
    batch2d = batch.reshape(_NBLK, 1, _BN)

    q0, k0, v0, s0 = _tc_qkvs(x, wc0, bc0)
    p0, d0 = _sc_edge(q0, k0, v0, src, dst)
    p0 = p0.reshape(2, N, 128)
    d0 = d0.reshape(32, _NBLK, _BN).transpose(1, 0, 2)
    q1, k1, v1, s1 = _tc_mid(p0, d0, s0, wc1, bc1)
    p1, d1 = _sc_edge(q1, k1, v1, src, dst)
    p1 = p1.reshape(2, N, 128)
    d1 = d1.reshape(32, _NBLK, _BN).transpose(1, 0, 2)
    out = _tc_final(p1, d1, s1, batch2d, linw_pad, linb_pad)
    return out[:, :NC]
